# baseline (speedup 1.0000x reference)
import functools

import jax
import jax.numpy as jnp
import numpy as np
from jax import lax
from jax.experimental import pallas as pl
from jax.experimental.pallas import tpu as pltpu
from jax.experimental.pallas import tpu_sc as plsc

MUXI = 4096
MUXO = 64
_MININT = np.int32(-2147483648)

BR = 8
HS = 2
HWB = 64 // HS
NB_IN = 4
NB_OUT = 2

_L = 16
_CH = MUXI // _L
_NC, _NS = 2, 16


def _pert_body(u_ref, logit_ref, o_ref):
    u = u_ref[...]
    gn = -jnp.log(-jnp.log(u + 1e-20) + 1e-20)
    o_ref[...] = logit_ref[...] + gn


_sc_mesh = plsc.VectorSubcoreMesh(core_axis_name="c", subcore_axis_name="s")


def _sc_mask_body(pert_hbm, mask_hbm, vbuf, kbuf, mbuf, hist):
    wid = lax.axis_index("s") * _NC + lax.axis_index("c")
    pltpu.sync_copy(pert_hbm.at[wid], vbuf)

    ones = jnp.ones((_L,), jnp.int32)
    minint = jnp.int32(_MININT)

    def key_chunk(i, _):
        v = vbuf[pl.ds(i * _L, _L)]
        r = plsc.bitcast(v, jnp.int32)
        k = r ^ (lax.shift_right_arithmetic(r, 31) & jnp.int32(0x7FFFFFFF))
        kbuf[pl.ds(i * _L, _L)] = k ^ minint
        return 0

    lax.fori_loop(0, _CH, key_chunk, 0)

    lane = lax.iota(jnp.int32, _L)
    rank = jnp.int32(MUXO)
    prefix = jnp.int32(0)

    for lvl in range(7, -1, -1):
        shift = 4 * lvl
        hist[...] = jnp.zeros((_L,), jnp.int32)

        if lvl == 7:
            def hpass(i, _, shift=shift):
                u = kbuf[pl.ds(i * _L, _L)]
                digit = lax.shift_right_logical(u, shift) & jnp.int32(0xF)
                plsc.addupdate_scatter(hist, [digit], ones)
                return 0
        else:
            def hpass(i, _, pfx=prefix, shift=shift):
                u = kbuf[pl.ds(i * _L, _L)]
                act = lax.shift_right_logical(u, shift + 4) == pfx
                digit = lax.shift_right_logical(u, shift) & jnp.int32(0xF)
                plsc.addupdate_scatter(hist, [digit], ones, mask=act)
                return 0

        lax.fori_loop(0, _CH, hpass, 0)

        h = hist[...]
        hr = lax.rev(h, (0,))
        cum = plsc.cumsum(hr)
        j = plsc.all_reduce_ffs(cum >= rank)
        d = jnp.int32(15) - j
        cum_j = jnp.max(jnp.where(lane == j, cum, 0))
        hr_j = jnp.max(jnp.where(lane == j, hr, 0))
        rank = rank - (cum_j - hr_j)
        prefix = (prefix << 4) | d

    t_u = prefix
    t_s = t_u ^ minint
    need = rank

    def mpass(i, run):
        u = kbuf[pl.ds(i * _L, _L)]
        s = u ^ minint
        gt = s > t_s
        eq = u == t_u
        ec = jnp.where(eq, 1, 0).astype(jnp.int32)
        c = plsc.cumsum(ec)
        sel = eq & ((run + c) <= need)
        mbuf[pl.ds(i * _L, _L)] = jnp.where(gt | sel, 1.0, 0.0)
        return run + jnp.max(c)

    lax.fori_loop(0, _CH, mpass, jnp.int32(0))
    pltpu.sync_copy(mbuf, mask_hbm.at[wid])


_sc_mask = functools.partial(
    pl.kernel,
    out_type=jax.ShapeDtypeStruct((_NC * _NS, MUXI), jnp.float32),
    mesh=_sc_mesh,
    compiler_params=pltpu.CompilerParams(needs_layout_passes=False),
    scratch_types=[
        pltpu.VMEM((MUXI,), jnp.float32),
        pltpu.VMEM((MUXI,), jnp.int32),
        pltpu.VMEM((MUXI,), jnp.float32),
        pltpu.VMEM((_L,), jnp.int32),
    ],
)(_sc_mask_body)


def _apply_body(mask_ref, x_hbm, o_hbm, xbuf, obuf, insem, outsem):
    nblk = (mask_ref.shape[0] // BR) * HS

    def in_copy(k, buf):
        b, h = divmod(k, HS)
        return pltpu.make_async_copy(
            x_hbm.at[pl.ds(b * BR, BR), pl.ds(h * HWB, HWB)],
            xbuf.at[buf], insem.at[buf])

    def out_copy(k, buf):
        b, h = divmod(k, HS)
        return pltpu.make_async_copy(
            obuf.at[buf],
            o_hbm.at[pl.ds(b * BR, BR), pl.ds(h * HWB, HWB)],
            outsem.at[buf])

    for k in range(NB_IN):
        in_copy(k, k).start()

    for k in range(nblk):
        in_copy(k, k % NB_IN).wait()
        b = k // HS
        m = mask_ref[pl.ds(b * BR, BR), :]
        y = xbuf[k % NB_IN] * m[:, None, :]
        if k >= NB_OUT:
            out_copy(k - NB_OUT, k % NB_OUT).wait()
        obuf[k % NB_OUT] = y
        out_copy(k, k % NB_OUT).start()
        if k + NB_IN < nblk:
            in_copy(k + NB_IN, k % NB_IN).start()

    for k in range(nblk - NB_OUT, nblk):
        out_copy(k, k % NB_OUT).wait()


def kernel(inputs, u, logits):
    bsz = inputs.shape[0]
    u2 = u.reshape(bsz, MUXI)
    x = inputs.reshape(bsz, 64, MUXI)

    pert = pl.pallas_call(
        _pert_body,
        out_shape=jax.ShapeDtypeStruct((bsz, MUXI), jnp.float32),
    )(u2, logits)

    mask = _sc_mask(pert)

    out = pl.pallas_call(
        _apply_body,
        in_specs=[
            pl.BlockSpec(memory_space=pltpu.VMEM),
            pl.BlockSpec(memory_space=pl.ANY),
        ],
        out_specs=pl.BlockSpec(memory_space=pl.ANY),
        out_shape=jax.ShapeDtypeStruct((bsz, 64, MUXI), jnp.float32),
        scratch_shapes=[
            pltpu.VMEM((NB_IN, BR, HWB, MUXI), jnp.float32),
            pltpu.VMEM((NB_OUT, BR, HWB, MUXI), jnp.float32),
            pltpu.SemaphoreType.DMA((NB_IN,)),
            pltpu.SemaphoreType.DMA((NB_OUT,)),
        ],
    )(mask, x)
    return out.reshape(inputs.shape)


# speedup vs baseline: 1.0251x; 1.0251x over previous
import functools

import jax
import jax.numpy as jnp
import numpy as np
from jax import lax
from jax.experimental import pallas as pl
from jax.experimental.pallas import tpu as pltpu
from jax.experimental.pallas import tpu_sc as plsc

MUXI = 4096
MUXO = 64
_MININT = np.int32(-2147483648)

BR = 8
HS = 2
HWB = 64 // HS
NB_IN = 4
NB_OUT = 2

_L = 16
_CH = MUXI // _L
_NC, _NS = 2, 16


def _pert_body(u_ref, logit_ref, o_ref):
    u = u_ref[...]
    gn = -jnp.log(-jnp.log(u + 1e-20) + 1e-20)
    o_ref[...] = logit_ref[...] + gn


_sc_mesh = plsc.VectorSubcoreMesh(core_axis_name="c", subcore_axis_name="s")


def _sc_mask_body(pert_hbm, mask_hbm, vbuf, kbuf, mbuf, hist):
    wid = lax.axis_index("s") * _NC + lax.axis_index("c")
    pltpu.sync_copy(pert_hbm.at[wid], vbuf)

    ones = jnp.ones((_L,), jnp.int32)
    minint = jnp.int32(_MININT)
    unroll = 8
    n_grp = _CH // unroll

    def key_chunk(g, _):
        for q in range(unroll):
            i = g * unroll + q
            v = vbuf[pl.ds(i * _L, _L)]
            r = plsc.bitcast(v, jnp.int32)
            k = r ^ (lax.shift_right_arithmetic(r, 31) & jnp.int32(0x7FFFFFFF))
            u = k ^ minint
            kbuf[pl.ds(i * _L, _L)] = u
            digit = lax.shift_right_logical(u, 28) & jnp.int32(0xF)
            plsc.addupdate_scatter(hist, [digit], ones)
        return 0

    hist[...] = jnp.zeros((_L,), jnp.int32)
    lax.fori_loop(0, n_grp, key_chunk, 0)

    lane = lax.iota(jnp.int32, _L)
    rank = jnp.int32(MUXO)
    prefix = jnp.int32(0)

    for lvl in range(7, -1, -1):
        shift = 4 * lvl

        if lvl < 7:
            def hpass(g, _, pfx=prefix, shift=shift):
                for q in range(unroll):
                    i = g * unroll + q
                    u = kbuf[pl.ds(i * _L, _L)]
                    act = lax.shift_right_logical(u, shift + 4) == pfx
                    digit = lax.shift_right_logical(u, shift) & jnp.int32(0xF)
                    plsc.addupdate_scatter(hist, [digit], ones, mask=act)
                return 0

            hist[...] = jnp.zeros((_L,), jnp.int32)
            lax.fori_loop(0, n_grp, hpass, 0)

        h = hist[...]
        hr = lax.rev(h, (0,))
        cum = plsc.cumsum(hr)
        j = plsc.all_reduce_ffs(cum >= rank)
        d = jnp.int32(15) - j
        cum_j = jnp.max(jnp.where(lane == j, cum, 0))
        hr_j = jnp.max(jnp.where(lane == j, hr, 0))
        rank = rank - (cum_j - hr_j)
        prefix = (prefix << 4) | d

    t_u = prefix
    t_s = t_u ^ minint
    need = rank

    def mpass(g, run):
        for q in range(unroll):
            i = g * unroll + q
            u = kbuf[pl.ds(i * _L, _L)]
            s = u ^ minint
            gt = s > t_s
            eq = u == t_u
            ec = jnp.where(eq, 1, 0).astype(jnp.int32)
            c = plsc.cumsum(ec)
            sel = eq & ((run + c) <= need)
            mbuf[pl.ds(i * _L, _L)] = jnp.where(gt | sel, 1.0, 0.0)
            run = run + jnp.max(c)
        return run

    lax.fori_loop(0, n_grp, mpass, jnp.int32(0))
    pltpu.sync_copy(mbuf, mask_hbm.at[wid])


_sc_mask = functools.partial(
    pl.kernel,
    out_type=jax.ShapeDtypeStruct((_NC * _NS, MUXI), jnp.float32),
    mesh=_sc_mesh,
    compiler_params=pltpu.CompilerParams(needs_layout_passes=False),
    scratch_types=[
        pltpu.VMEM((MUXI,), jnp.float32),
        pltpu.VMEM((MUXI,), jnp.int32),
        pltpu.VMEM((MUXI,), jnp.float32),
        pltpu.VMEM((_L,), jnp.int32),
    ],
)(_sc_mask_body)


def _apply_body(mask_ref, x_hbm, o_hbm, xbuf, obuf, insem, outsem):
    nblk = (mask_ref.shape[0] // BR) * HS

    def in_copy(k, buf):
        b, h = divmod(k, HS)
        return pltpu.make_async_copy(
            x_hbm.at[pl.ds(b * BR, BR), pl.ds(h * HWB, HWB)],
            xbuf.at[buf], insem.at[buf])

    def out_copy(k, buf):
        b, h = divmod(k, HS)
        return pltpu.make_async_copy(
            obuf.at[buf],
            o_hbm.at[pl.ds(b * BR, BR), pl.ds(h * HWB, HWB)],
            outsem.at[buf])

    for k in range(NB_IN):
        in_copy(k, k).start()

    for k in range(nblk):
        in_copy(k, k % NB_IN).wait()
        b = k // HS
        m = mask_ref[pl.ds(b * BR, BR), :]
        y = xbuf[k % NB_IN] * m[:, None, :]
        if k >= NB_OUT:
            out_copy(k - NB_OUT, k % NB_OUT).wait()
        obuf[k % NB_OUT] = y
        out_copy(k, k % NB_OUT).start()
        if k + NB_IN < nblk:
            in_copy(k + NB_IN, k % NB_IN).start()

    for k in range(nblk - NB_OUT, nblk):
        out_copy(k, k % NB_OUT).wait()


def kernel(inputs, u, logits):
    bsz = inputs.shape[0]
    u2 = u.reshape(bsz, MUXI)
    x = inputs.reshape(bsz, 64, MUXI)

    pert = pl.pallas_call(
        _pert_body,
        out_shape=jax.ShapeDtypeStruct((bsz, MUXI), jnp.float32),
    )(u2, logits)

    mask = _sc_mask(pert)

    out = pl.pallas_call(
        _apply_body,
        in_specs=[
            pl.BlockSpec(memory_space=pltpu.VMEM),
            pl.BlockSpec(memory_space=pl.ANY),
        ],
        out_specs=pl.BlockSpec(memory_space=pl.ANY),
        out_shape=jax.ShapeDtypeStruct((bsz, 64, MUXI), jnp.float32),
        scratch_shapes=[
            pltpu.VMEM((NB_IN, BR, HWB, MUXI), jnp.float32),
            pltpu.VMEM((NB_OUT, BR, HWB, MUXI), jnp.float32),
            pltpu.SemaphoreType.DMA((NB_IN,)),
            pltpu.SemaphoreType.DMA((NB_OUT,)),
        ],
    )(mask, x)
    return out.reshape(inputs.shape)


# speedup vs baseline: 1.0479x; 1.0223x over previous
import functools

import jax
import jax.numpy as jnp
import numpy as np
from jax import lax
from jax.experimental import pallas as pl
from jax.experimental.pallas import tpu as pltpu
from jax.experimental.pallas import tpu_sc as plsc

MUXI = 4096
MUXO = 64
_MININT = np.int32(-2147483648)

BR = 8
HS = 2
HWB = 64 // HS
NB_IN = 4
NB_OUT = 2

_L = 16
_CH = MUXI // _L
_NC, _NS = 2, 16


def _pert_body(u_ref, logit_ref, o_ref):
    u = u_ref[...]
    gn = -jnp.log(-jnp.log(u + 1e-20) + 1e-20)
    o_ref[...] = logit_ref[...] + gn


_sc_mesh = plsc.VectorSubcoreMesh(core_axis_name="c", subcore_axis_name="s")


def _sc_mask_body(pert_hbm, mask_hbm, vbuf, kbuf, mbuf, h0, h1, h2, h3):
    wid = lax.axis_index("s") * _NC + lax.axis_index("c")
    pltpu.sync_copy(pert_hbm.at[wid], vbuf)

    hists = (h0, h1, h2, h3)
    ones = jnp.ones((_L,), jnp.int32)
    minint = jnp.int32(_MININT)
    unroll = 8
    n_grp = _CH // unroll

    def zero_hists():
        for h in hists:
            h[...] = jnp.zeros((_L,), jnp.int32)

    def key_chunk(g, _):
        for q in range(unroll):
            i = g * unroll + q
            v = vbuf[pl.ds(i * _L, _L)]
            r = plsc.bitcast(v, jnp.int32)
            k = r ^ (lax.shift_right_arithmetic(r, 31) & jnp.int32(0x7FFFFFFF))
            u = k ^ minint
            kbuf[pl.ds(i * _L, _L)] = u
            digit = lax.shift_right_logical(u, 28) & jnp.int32(0xF)
            plsc.addupdate_scatter(hists[q % 4], [digit], ones)
        return 0

    zero_hists()
    lax.fori_loop(0, n_grp, key_chunk, 0)

    lane = lax.iota(jnp.int32, _L)
    rank = jnp.int32(MUXO)
    prefix = jnp.int32(0)
    cnt_eq = jnp.int32(0)

    for lvl in range(7, -1, -1):
        shift = 4 * lvl

        if lvl < 7:
            def hpass(g, _, pfx=prefix, shift=shift):
                for q in range(unroll):
                    i = g * unroll + q
                    u = kbuf[pl.ds(i * _L, _L)]
                    act = lax.shift_right_logical(u, shift + 4) == pfx
                    digit = lax.shift_right_logical(u, shift) & jnp.int32(0xF)
                    plsc.addupdate_scatter(hists[q % 4], [digit], ones,
                                           mask=act)
                return 0

            zero_hists()
            lax.fori_loop(0, n_grp, hpass, 0)

        h = h0[...] + h1[...] + h2[...] + h3[...]
        hr = lax.rev(h, (0,))
        cum = plsc.cumsum(hr)
        j = plsc.all_reduce_ffs(cum >= rank)
        d = jnp.int32(15) - j
        cum_j = jnp.max(jnp.where(lane == j, cum, 0))
        hr_j = jnp.max(jnp.where(lane == j, hr, 0))
        rank = rank - (cum_j - hr_j)
        prefix = (prefix << 4) | d
        cnt_eq = hr_j

    t_u = prefix
    t_s = t_u ^ minint
    need = rank

    @pl.when(cnt_eq == need)
    def _():
        def mpass(g, _):
            for q in range(unroll):
                i = g * unroll + q
                s = kbuf[pl.ds(i * _L, _L)] ^ minint
                mbuf[pl.ds(i * _L, _L)] = jnp.where(s >= t_s, 1.0, 0.0)
            return 0

        lax.fori_loop(0, n_grp, mpass, 0)

    @pl.when(cnt_eq != need)
    def _():
        def mpass(g, run):
            for q in range(unroll):
                i = g * unroll + q
                u = kbuf[pl.ds(i * _L, _L)]
                s = u ^ minint
                gt = s > t_s
                eq = u == t_u
                ec = jnp.where(eq, 1, 0).astype(jnp.int32)
                c = plsc.cumsum(ec)
                sel = eq & ((run + c) <= need)
                mbuf[pl.ds(i * _L, _L)] = jnp.where(gt | sel, 1.0, 0.0)
                run = run + jnp.max(c)
            return run

        lax.fori_loop(0, n_grp, mpass, jnp.int32(0))

    pltpu.sync_copy(mbuf, mask_hbm.at[wid])


_sc_mask = functools.partial(
    pl.kernel,
    out_type=jax.ShapeDtypeStruct((_NC * _NS, MUXI), jnp.float32),
    mesh=_sc_mesh,
    compiler_params=pltpu.CompilerParams(needs_layout_passes=False),
    scratch_types=[
        pltpu.VMEM((MUXI,), jnp.float32),
        pltpu.VMEM((MUXI,), jnp.int32),
        pltpu.VMEM((MUXI,), jnp.float32),
        pltpu.VMEM((_L,), jnp.int32),
        pltpu.VMEM((_L,), jnp.int32),
        pltpu.VMEM((_L,), jnp.int32),
        pltpu.VMEM((_L,), jnp.int32),
    ],
)(_sc_mask_body)


def _apply_body(mask_ref, x_hbm, o_hbm, xbuf, obuf, insem, outsem):
    nblk = (mask_ref.shape[0] // BR) * HS

    def in_copy(k, buf):
        b, h = divmod(k, HS)
        return pltpu.make_async_copy(
            x_hbm.at[pl.ds(b * BR, BR), pl.ds(h * HWB, HWB)],
            xbuf.at[buf], insem.at[buf])

    def out_copy(k, buf):
        b, h = divmod(k, HS)
        return pltpu.make_async_copy(
            obuf.at[buf],
            o_hbm.at[pl.ds(b * BR, BR), pl.ds(h * HWB, HWB)],
            outsem.at[buf])

    for k in range(NB_IN):
        in_copy(k, k).start()

    for k in range(nblk):
        in_copy(k, k % NB_IN).wait()
        b = k // HS
        m = mask_ref[pl.ds(b * BR, BR), :]
        y = xbuf[k % NB_IN] * m[:, None, :]
        if k >= NB_OUT:
            out_copy(k - NB_OUT, k % NB_OUT).wait()
        obuf[k % NB_OUT] = y
        out_copy(k, k % NB_OUT).start()
        if k + NB_IN < nblk:
            in_copy(k + NB_IN, k % NB_IN).start()

    for k in range(nblk - NB_OUT, nblk):
        out_copy(k, k % NB_OUT).wait()


def kernel(inputs, u, logits):
    bsz = inputs.shape[0]
    u2 = u.reshape(bsz, MUXI)
    x = inputs.reshape(bsz, 64, MUXI)

    pert = pl.pallas_call(
        _pert_body,
        out_shape=jax.ShapeDtypeStruct((bsz, MUXI), jnp.float32),
    )(u2, logits)

    mask = _sc_mask(pert)

    out = pl.pallas_call(
        _apply_body,
        in_specs=[
            pl.BlockSpec(memory_space=pltpu.VMEM),
            pl.BlockSpec(memory_space=pl.ANY),
        ],
        out_specs=pl.BlockSpec(memory_space=pl.ANY),
        out_shape=jax.ShapeDtypeStruct((bsz, 64, MUXI), jnp.float32),
        scratch_shapes=[
            pltpu.VMEM((NB_IN, BR, HWB, MUXI), jnp.float32),
            pltpu.VMEM((NB_OUT, BR, HWB, MUXI), jnp.float32),
            pltpu.SemaphoreType.DMA((NB_IN,)),
            pltpu.SemaphoreType.DMA((NB_OUT,)),
        ],
    )(mask, x)
    return out.reshape(inputs.shape)


# speedup vs baseline: 2.1078x; 2.0114x over previous
import jax
import jax.numpy as jnp
import numpy as np
from jax.experimental import pallas as pl
from jax.experimental.pallas import tpu as pltpu

MUXI = 4096
MUXO = 64
_MININT = np.int32(-2147483648)

BR = 8
HS = 4
HWB = 64 // HS
NB_IN = 8
NB_OUT = 4


def _count(m):
    mf = jnp.where(m, 1.0, 0.0).astype(jnp.float32)
    ones = jnp.full((MUXI, 128), 1.0, jnp.float32)
    c = jax.lax.dot_general(mf, ones, (((1,), (0,)), ((), ())),
                            preferred_element_type=jnp.float32)
    return c[:, :1]


def _write_mask(u, logits, mask_ref):
    gn = -jnp.log(-jnp.log(u + 1e-20) + 1e-20)
    pert = logits + gn

    raw = jax.lax.bitcast_convert_type(pert, jnp.int32)
    key = raw ^ (jax.lax.shift_right_arithmetic(raw, 31) & jnp.int32(0x7FFFFFFF))

    bsz = u.shape[0]
    kcnt = jnp.float32(MUXO)

    def bit_step(b, t_u):
        shift = 31 - b
        cand = t_u | jax.lax.shift_left(jnp.int32(1), shift)
        thr = cand ^ _MININT
        cnt = _count(key >= thr)
        return jnp.where(cnt >= kcnt, cand, t_u)

    t_u = jax.lax.fori_loop(0, 32, bit_step, jnp.zeros((bsz, 1), jnp.int32))
    thr = t_u ^ _MININT

    gt = key > thr
    eq = key == thr
    c_ge = _count(gt | eq)
    ties = jnp.max(c_ge) > kcnt

    @pl.when(jnp.logical_not(ties))
    def _():
        mask_ref[...] = jnp.where(gt | eq, 1.0, 0.0).astype(jnp.float32)

    @pl.when(ties)
    def _():
        need = kcnt - _count(gt)
        idx = jax.lax.broadcasted_iota(jnp.int32, key.shape, 1)

        def bit_step2(b, sel_j):
            shift = 12 - b
            cand = sel_j | jax.lax.shift_left(jnp.int32(1), shift)
            cnt = _count(eq & (idx < cand))
            return jnp.where(cnt <= need, cand, sel_j)

        sel_j = jax.lax.fori_loop(0, 13, bit_step2,
                                  jnp.zeros((bsz, 1), jnp.int32))
        mask = gt | (eq & (idx < sel_j))
        mask_ref[...] = jnp.where(mask, 1.0, 0.0).astype(jnp.float32)


def _body(u_ref, logit_ref, x_hbm, o_hbm, mask_ref, xbuf, obuf, insem, outsem):
    nblk = (u_ref.shape[0] // BR) * HS

    def in_copy(k, buf):
        b, h = divmod(k, HS)
        return pltpu.make_async_copy(
            x_hbm.at[pl.ds(b * BR, BR), pl.ds(h * HWB, HWB)],
            xbuf.at[buf], insem.at[buf])

    def out_copy(k, buf):
        b, h = divmod(k, HS)
        return pltpu.make_async_copy(
            obuf.at[buf],
            o_hbm.at[pl.ds(b * BR, BR), pl.ds(h * HWB, HWB)],
            outsem.at[buf])

    for k in range(NB_IN):
        in_copy(k, k).start()

    _write_mask(u_ref[...], logit_ref[...], mask_ref)

    for k in range(nblk):
        in_copy(k, k % NB_IN).wait()
        b = k // HS
        m = mask_ref[pl.ds(b * BR, BR), :]
        y = xbuf[k % NB_IN] * m[:, None, :]
        if k >= NB_OUT:
            out_copy(k - NB_OUT, k % NB_OUT).wait()
        obuf[k % NB_OUT] = y
        out_copy(k, k % NB_OUT).start()
        if k + NB_IN < nblk:
            in_copy(k + NB_IN, k % NB_IN).start()

    for k in range(nblk - NB_OUT, nblk):
        out_copy(k, k % NB_OUT).wait()


def kernel(inputs, u, logits):
    bsz = inputs.shape[0]
    u2 = u.reshape(bsz, MUXI)
    x = inputs.reshape(bsz, 64, MUXI)

    out = pl.pallas_call(
        _body,
        in_specs=[
            pl.BlockSpec(memory_space=pltpu.VMEM),
            pl.BlockSpec(memory_space=pltpu.VMEM),
            pl.BlockSpec(memory_space=pl.ANY),
        ],
        out_specs=pl.BlockSpec(memory_space=pl.ANY),
        out_shape=jax.ShapeDtypeStruct((bsz, 64, MUXI), jnp.float32),
        scratch_shapes=[
            pltpu.VMEM((bsz, MUXI), jnp.float32),
            pltpu.VMEM((NB_IN, BR, HWB, MUXI), jnp.float32),
            pltpu.VMEM((NB_OUT, BR, HWB, MUXI), jnp.float32),
            pltpu.SemaphoreType.DMA((NB_IN,)),
            pltpu.SemaphoreType.DMA((NB_OUT,)),
        ],
    )(u2, logits, x)
    return out.reshape(inputs.shape)
